# Initial kernel scaffold; baseline (speedup 1.0000x reference)
import jax
import jax.numpy as jnp
from jax.experimental import pallas as pl


def kernel(V_matrix, P_matrix, raw_base_points):
    raise NotImplementedError("write your pallas kernel here")


# speedup vs baseline: 11.7299x; 11.7299x over previous
import jax
import jax.numpy as jnp
from jax.experimental import pallas as pl

N = 13860
NP = 14336
ROWS = 112
H = 512
W = 512
OUT = 517


def _rank_body(zc_ref, zr_ref, mc_ref, aux_ref, out_ref):
    zi = zc_ref[0]
    acc = jnp.zeros((128, 128), jnp.float32)
    for r in range(ROWS):
        rowv = zr_ref[0, r : r + 1, :]
        acc = acc + (rowv < zi).astype(jnp.float32)
    c = jnp.sum(acc, axis=1, keepdims=True)
    a = aux_ref[0]
    denom = a[0:1, 0:1]
    nfg = a[0:1, 1:2]
    scaled = (1.0 - c / denom) * 0.6 + 0.2
    val = jnp.where(nfg > 1.5, scaled, 0.5)
    m = mc_ref[0]
    out_ref[0] = jnp.where(m > 0.5, val, 1.0)


def _erode_body(img_ref, out_ref):
    x = img_ref[0]
    mid = jnp.concatenate(
        [
            jnp.full((512, 8), 1.0, jnp.float32),
            x,
            jnp.full((512, 120), 1.0, jnp.float32),
        ],
        axis=1,
    )
    buf = jnp.concatenate(
        [
            jnp.full((8, 640), 1.0, jnp.float32),
            mid,
            jnp.full((8, 640), 1.0, jnp.float32),
        ],
        axis=0,
    )
    one_r1 = jnp.full((1, 640), 1.0, jnp.float32)
    one_r2 = jnp.full((2, 640), 1.0, jnp.float32)
    one_c1 = jnp.full((528, 1), 1.0, jnp.float32)
    one_c2 = jnp.full((528, 2), 1.0, jnp.float32)
    for _ in range(5):
        dn1 = jnp.concatenate([one_r1, buf[:-1]], axis=0)
        up1 = jnp.concatenate([buf[1:], one_r1], axis=0)
        up2 = jnp.concatenate([buf[2:], one_r2], axis=0)
        buf = jnp.minimum(jnp.minimum(dn1, buf), jnp.minimum(up1, up2))
        le1 = jnp.concatenate([one_c1, buf[:, :-1]], axis=1)
        ri1 = jnp.concatenate([buf[:, 1:], one_c1], axis=1)
        ri2 = jnp.concatenate([buf[:, 2:], one_c2], axis=1)
        buf = jnp.minimum(jnp.minimum(le1, buf), jnp.minimum(ri1, ri2))
    out_ref[0] = jax.lax.slice(buf, (3, 3), (3 + OUT, 3 + OUT))


def kernel(V_matrix, P_matrix, raw_base_points):
    B = V_matrix.shape[0]
    VP = jnp.matmul(P_matrix, V_matrix)
    points = jnp.broadcast_to(
        raw_base_points[None], (B,) + raw_base_points.shape
    )
    tph = jnp.matmul(points, jnp.swapaxes(VP, 1, 2))
    wq = tph[..., 3:4]
    ndc = jnp.where(wq != 0, tph[..., :3] / wq, tph[..., :3])
    sx = (ndc[..., 0] + 1.0) * 0.5 * W
    sy = (1.0 - (ndc[..., 1] + 1.0) * 0.5) * H
    xi = jnp.round(sx).astype(jnp.int32)
    yi = jnp.round(sy).astype(jnp.int32)
    valid = (xi >= 0) & (xi < W) & (yi >= 0) & (yi < H)
    xc = jnp.clip(xi, 0, W - 1)
    yc = jnp.clip(yi, 0, H - 1)
    p_live = yc * W + xc
    z_live = jnp.where(valid, ndc[..., 2], 2.0)

    pad_p = jnp.zeros((B, NP - N), jnp.int32)
    pad_z = jnp.full((B, NP - N), 2.0, jnp.float32)
    p = jnp.concatenate([p_live, pad_p], axis=1)
    z = jnp.concatenate([z_live, pad_z], axis=1)

    bidx = jnp.arange(B)[:, None]
    zb = jnp.full((B, H * W), 2.0, jnp.float32).at[bidx, p].min(z)
    zb_at = jnp.take_along_axis(zb, p, axis=1)
    m = (z < 2.0) & (z == zb_at)
    nfg = jnp.sum(m, axis=1).astype(jnp.float32)
    denom = jnp.maximum(nfg - 1.0, 1.0)
    aux = (
        jnp.zeros((B, 1, 128), jnp.float32)
        .at[:, 0, 0]
        .set(denom)
        .at[:, 0, 1]
        .set(nfg)
    )
    zkey = jnp.where(m, z, 4.0)

    vals = pl.pallas_call(
        _rank_body,
        grid=(B, NP // 128),
        in_specs=[
            pl.BlockSpec((1, 128, 1), lambda b, i: (b, i, 0)),
            pl.BlockSpec((1, ROWS, 128), lambda b, i: (b, 0, 0)),
            pl.BlockSpec((1, 128, 1), lambda b, i: (b, i, 0)),
            pl.BlockSpec((1, 1, 128), lambda b, i: (b, 0, 0)),
        ],
        out_specs=pl.BlockSpec((1, 128, 1), lambda b, i: (b, i, 0)),
        out_shape=jax.ShapeDtypeStruct((B, NP, 1), jnp.float32),
    )(
        zkey.reshape(B, NP, 1),
        zkey.reshape(B, ROWS, 128),
        m.astype(jnp.float32).reshape(B, NP, 1),
        aux,
    )

    img = (
        jnp.full((B, H * W), 1.0, jnp.float32)
        .at[bidx, p]
        .min(vals.reshape(B, NP))
        .reshape(B, H, W)
    )

    out = pl.pallas_call(
        _erode_body,
        grid=(B,),
        in_specs=[pl.BlockSpec((1, H, W), lambda b: (b, 0, 0))],
        out_specs=pl.BlockSpec((1, OUT, OUT), lambda b: (b, 0, 0)),
        out_shape=jax.ShapeDtypeStruct((B, OUT, OUT), jnp.float32),
    )(img)

    return out[:, None]
